# baseline (speedup 1.0000x reference)
import jax
import jax.numpy as jnp
from jax.experimental import pallas as pl

_N_CLASSES = 1000
_B0, _B1 = 1024, 20
_ROWS = 128


def _onehot_body(idx_ref, out_ref):
    iota = jax.lax.broadcasted_iota(jnp.int32, out_ref.shape, 2)
    out_ref[...] = (iota == idx_ref[...][:, :, None]).astype(jnp.float32)


def kernel(input):
    idx = input.astype(jnp.int32)
    return pl.pallas_call(
        _onehot_body,
        grid=(_B0 // _ROWS,),
        in_specs=[pl.BlockSpec((_ROWS, _B1), lambda i: (i, 0))],
        out_specs=pl.BlockSpec((_ROWS, _B1, _N_CLASSES), lambda i: (i, 0, 0)),
        out_shape=jax.ShapeDtypeStruct((_B0, _B1, _N_CLASSES), jnp.float32),
    )(idx)
